# baseline (speedup 1.0000x reference)
import functools

import jax
import jax.numpy as jnp
from jax import lax
from jax.experimental import pallas as pl
from jax.experimental.pallas import tpu as pltpu
from jax.experimental.pallas import tpu_sc as plsc

N_NODES = 10000
N_EDGES = 320000
NFEAT = 256
NHID = 128
NCLASS = 64

N_TILES = 16
CH = 128
N_PAD = 10240
ROWS_PER_TILE = N_PAD // N_TILES
E_PAD = 327680
CHUNKS = E_PAD // N_TILES // CH


def _make_sc_agg(feat_half, with_deg, mode, group, tc_tiling=True):
  mesh = plsc.VectorSubcoreMesh(core_axis_name="c", subcore_axis_name="s")
  G = group
  chunks = CHUNKS if mode == "feat" else CHUNKS // 2
  ngroups = chunks // G
  assert ngroups * G == chunks and ngroups % 2 == 0
  out_type = [jax.ShapeDtypeStruct((2, N_PAD, feat_half), jnp.float32)]
  if with_deg:
    out_type.append(jax.ShapeDtypeStruct((2, N_TILES, N_PAD), jnp.float32))
  scratch = [
      pltpu.VMEM((G, CH), jnp.int32),
      pltpu.VMEM((G, CH), jnp.int32),
      pltpu.VMEM((G, CH), jnp.int32),
      pltpu.VMEM((G, CH), jnp.int32),
      pltpu.VMEM((CH, feat_half), jnp.float32),
      pltpu.VMEM((CH, feat_half), jnp.float32),
      pltpu.VMEM_SHARED((N_PAD, feat_half), jnp.float32),
      pltpu.SemaphoreType.DMA,
      pltpu.SemaphoreType.DMA,
      pltpu.SemaphoreType.DMA,
      pltpu.SemaphoreType.DMA,
      pltpu.SemaphoreType.DMA,
      pltpu.SemaphoreType.DMA,
  ]
  if with_deg:
    scratch.append(pltpu.VMEM((N_PAD,), jnp.float32))

  def body(xs, srcs, dsts, zrows, *rest):
    degp = None
    if with_deg:
      zdeg, agg_out, deg_out = rest[0], rest[1], rest[2]
      (sa, da, sb, db, rows_a, rows_b, agg_sh,
       gsem_a, gsem_b, ssem_a, ssem_b, isem_a, isem_b, degp) = rest[3:]
    else:
      agg_out = rest[0]
      (sa, da, sb, db, rows_a, rows_b, agg_sh,
       gsem_a, gsem_b, ssem_a, ssem_b, isem_a, isem_b) = rest[1:]
    c = lax.axis_index("c")
    s = lax.axis_index("s")
    row0 = s * ROWS_PER_TILE

    idx = ((sa, da, isem_a), (sb, db, isem_b))
    ring = ((rows_a, gsem_a, ssem_a), (rows_b, gsem_b, ssem_b))
    if mode == "feat":
      cbase = s * ngroups
    else:
      cbase = (c * N_TILES + s) * ngroups

    def start_idx(g, b):
      off = pl.multiple_of((cbase + g) * G, 8)
      sbuf, dbuf, isem = idx[b]
      pltpu.async_copy(srcs.at[pl.ds(off, G)], sbuf, isem)
      pltpu.async_copy(dsts.at[pl.ds(off, G)], dbuf, isem)

    def wait_idx(b):
      sbuf, dbuf, isem = idx[b]
      pltpu.make_async_copy(srcs.at[pl.ds(0, G)], sbuf, isem).wait()
      pltpu.make_async_copy(dsts.at[pl.ds(0, G)], dbuf, isem).wait()

    def table(idx_row):
      if mode == "feat":
        return xs.at[c].at[idx_row]
      return xs.at[idx_row]

    def gather(idx_row, buf, sem):
      pltpu.async_copy(table(idx_row), buf, sem)

    def gwait(idx_row, buf, sem):
      pltpu.make_async_copy(table(idx_row), buf, sem).wait()

    def scat(idx_row, buf, ssem):
      pltpu.async_copy(buf, agg_sh.at[idx_row], ssem, add=True)

    def swait(idx_row, buf, ssem):
      pltpu.make_async_copy(buf, agg_sh.at[idx_row], ssem).wait()

    pltpu.sync_copy(zrows, agg_sh.at[pl.ds(row0, ROWS_PER_TILE)])
    if with_deg:
      pltpu.sync_copy(zdeg, degp)
    ones16 = jnp.ones((16,), jnp.float32)

    start_idx(0, 0)
    start_idx(1, 1)
    pltpu.sync_copy(zrows.at[pl.ds(0, CH)], rows_b)
    plsc.subcore_barrier()
    wait_idx(0)
    scat(da.at[0], rows_b, ssem_b)
    gather(sa.at[0], rows_a, gsem_a)

    def do_group(b):
      sbuf, dbuf, _ = idx[b]
      nsbuf, ndbuf, _ = idx[1 - b]
      for k in range(G):
        buf, gsem, ssem = ring[k % 2]
        nbuf, ngsem, nssem = ring[(k + 1) % 2]
        gwait(sbuf.at[k], buf, gsem)
        scat(dbuf.at[k], buf, ssem)
        swait(dbuf.at[k], nbuf, nssem)
        if k == G - 1:
          wait_idx(1 - b)
          gather(nsbuf.at[0], nbuf, ngsem)
        else:
          gather(sbuf.at[k + 1], nbuf, ngsem)
        if with_deg:
          @pl.when(c == b)
          def _():
            for j in range(CH // 16):
              dvec = dbuf[k, pl.ds(j * 16, 16)]
              plsc.addupdate_scatter(degp, [dvec], ones16)

    def pair(p, carry):
      do_group(0)
      start_idx(jnp.minimum(2 * p + 2, ngroups - 1), 0)
      do_group(1)
      start_idx(jnp.minimum(2 * p + 3, ngroups - 1), 1)
      return carry

    lax.fori_loop(0, ngroups // 2, pair, 0)
    gwait(sa.at[0], rows_a, gsem_a)
    swait(da.at[0], rows_b, ssem_b)
    wait_idx(1)
    plsc.subcore_barrier()

    pltpu.sync_copy(agg_sh.at[pl.ds(row0, ROWS_PER_TILE)],
                    agg_out.at[c].at[pl.ds(row0, ROWS_PER_TILE)])
    if with_deg:
      pltpu.sync_copy(degp, deg_out.at[c].at[s])

  params = pltpu.CompilerParams(use_tc_tiling_on_sc=tc_tiling,
                                needs_layout_passes=not with_deg)
  return pl.kernel(body, out_type=out_type, mesh=mesh, scratch_types=scratch,
                   compiler_params=params)


_R1 = 1000


def _dense1_body(agg_ref, x_ref, deg_ref, w_ref, wcls_ref, out_ref, y_ref):
  agg = agg_ref[...]
  h = jnp.concatenate([agg[0], agg[1]], axis=1) + x_ref[...]
  r = 1.0 / (deg_ref[...] + 1.0)
  h = h * r
  acc = lax.dot_general(h, w_ref[...], (((1,), (1,)), ((), ())),
                        preferred_element_type=jnp.float32)
  m = jnp.maximum(acc, 0.0)
  out_ref[...] = m
  y_ref[...] = lax.dot_general(m[:, :NHID], wcls_ref[...],
                               (((1,), (1,)), ((), ())),
                               preferred_element_type=jnp.float32)


def _dense1(agg, x, degc, wcat, w_cls):
  return pl.pallas_call(
      _dense1_body,
      grid=(N_NODES // _R1,),
      in_specs=[
          pl.BlockSpec((2, _R1, NHID), lambda i: (0, i, 0)),
          pl.BlockSpec((_R1, NFEAT), lambda i: (i, 0)),
          pl.BlockSpec((_R1, 1), lambda i: (i, 0)),
          pl.BlockSpec((2 * NHID, NFEAT), lambda i: (0, 0)),
          pl.BlockSpec((NCLASS, NHID), lambda i: (0, 0)),
      ],
      out_specs=[
          pl.BlockSpec((_R1, 2 * NHID), lambda i: (i, 0)),
          pl.BlockSpec((_R1, NCLASS), lambda i: (i, 0)),
      ],
      out_shape=[
          jax.ShapeDtypeStruct((N_NODES, 2 * NHID), jnp.float32),
          jax.ShapeDtypeStruct((N_NODES, NCLASS), jnp.float32),
      ],
  )(agg, x, degc, wcat, w_cls)


def _dense2_body(agg_ref, y_ref, deg_ref, out_ref):
  agg = agg_ref[...]
  h = agg[0] + agg[1] + y_ref[...]
  r = 1.0 / (deg_ref[...] + 1.0)
  out_ref[...] = jnp.maximum(h * r, 0.0)


def _dense2(agg2, y, degc):
  return pl.pallas_call(
      _dense2_body,
      grid=(N_NODES // _R1,),
      in_specs=[
          pl.BlockSpec((2, _R1, NCLASS), lambda i: (0, i, 0)),
          pl.BlockSpec((_R1, NCLASS), lambda i: (i, 0)),
          pl.BlockSpec((_R1, 1), lambda i: (i, 0)),
      ],
      out_specs=pl.BlockSpec((_R1, NCLASS), lambda i: (i, 0)),
      out_shape=jax.ShapeDtypeStruct((N_NODES, NCLASS), jnp.float32),
  )(agg2, y, degc)


_RB = 256


def _recons_body(a_ref, b_ref, out_ref):
  z = lax.dot_general(a_ref[...], b_ref[...], (((1,), (1,)), ((), ())),
                      preferred_element_type=jnp.float32)
  out_ref[...] = 1.0 / (1.0 + jnp.exp(-z))


def _recons(mu):
  return pl.pallas_call(
      _recons_body,
      grid=(pl.cdiv(N_NODES, _RB),),
      in_specs=[
          pl.BlockSpec((_RB, NHID), lambda i: (i, 0)),
          pl.BlockSpec((N_NODES, NHID), lambda i: (0, 0)),
      ],
      out_specs=pl.BlockSpec((_RB, N_NODES), lambda i: (i, 0)),
      out_shape=jax.ShapeDtypeStruct((N_NODES, N_NODES), jnp.float32),
  )(mu, mu)


def kernel(x, edge_index, W_mu, W_logvar, W_cls):
  src = edge_index[0].astype(jnp.int32)
  dst = edge_index[1].astype(jnp.int32)
  npad_e = E_PAD - N_EDGES
  pad_idx = jnp.full((npad_e,), N_PAD - 1, jnp.int32)
  src_p = jnp.concatenate([src, pad_idx]).reshape(E_PAD // CH, CH)
  dst_p = jnp.concatenate([dst, pad_idx]).reshape(E_PAD // CH, CH)

  x_pad = jnp.zeros((N_PAD, NFEAT), jnp.float32).at[:N_NODES].set(x)
  xs = jnp.stack([x_pad[:, :NHID], x_pad[:, NHID:]])
  zrows1 = jnp.zeros((ROWS_PER_TILE, NHID), jnp.float32)
  zdeg = jnp.zeros((N_PAD,), jnp.float32)

  sc1 = _make_sc_agg(NHID, with_deg=True, mode="feat", group=8,
                     tc_tiling=False)
  agg, deg = sc1(xs, src_p, dst_p, zrows1, zdeg)
  degq = jnp.sum(deg, axis=(0, 1))[:N_NODES, None]

  wcat = jnp.concatenate([W_mu, W_logvar], axis=0)
  ml, y = _dense1(agg, x, degq, wcat, W_cls)
  mu = ml[:, :NHID]
  logvar = ml[:, NHID:]

  y_pad = jnp.zeros((N_PAD, NCLASS), jnp.float32).at[:N_NODES].set(y)
  zrows2 = jnp.zeros((ROWS_PER_TILE, NCLASS), jnp.float32)

  sc2 = _make_sc_agg(NCLASS, with_deg=False, mode="edge", group=8,
                     tc_tiling=False)
  agg2 = sc2(y_pad, src_p, dst_p, zrows2)
  if isinstance(agg2, (tuple, list)):
    agg2 = agg2[0]

  rst = _dense2(agg2, y, degq)
  recons = _recons(mu)
  return (rst, recons, mu, logvar)


# speedup vs baseline: 1.2295x; 1.2295x over previous
import functools

import jax
import jax.numpy as jnp
from jax import lax
from jax.experimental import pallas as pl
from jax.experimental.pallas import tpu as pltpu
from jax.experimental.pallas import tpu_sc as plsc

N_NODES = 10000
N_EDGES = 320000
NFEAT = 256
NHID = 128
NCLASS = 64

N_TILES = 16
CH = 128
N_PAD = 10240
ROWS_PER_TILE = N_PAD // N_TILES
E_PAD = 327680
CHUNKS = E_PAD // N_TILES // CH


def _make_sc_agg(feat_half, with_deg, mode, group, tc_tiling=True,
                 stage_table=False):
  mesh = plsc.VectorSubcoreMesh(core_axis_name="c", subcore_axis_name="s")
  G = group
  chunks = CHUNKS if mode == "feat" else CHUNKS // 2
  ngroups = chunks // G
  assert ngroups * G == chunks and ngroups % 2 == 0
  out_type = [jax.ShapeDtypeStruct((2, N_PAD, feat_half), jnp.float32)]
  if with_deg:
    out_type.append(jax.ShapeDtypeStruct((2, N_TILES, N_PAD), jnp.float32))
  scratch = [
      pltpu.VMEM((G, CH), jnp.int32),
      pltpu.VMEM((G, CH), jnp.int32),
      pltpu.VMEM((G, CH), jnp.int32),
      pltpu.VMEM((G, CH), jnp.int32),
      pltpu.VMEM((CH, feat_half), jnp.float32),
      pltpu.VMEM((CH, feat_half), jnp.float32),
      pltpu.VMEM_SHARED((N_PAD, feat_half), jnp.float32),
      pltpu.SemaphoreType.DMA,
      pltpu.SemaphoreType.DMA,
      pltpu.SemaphoreType.DMA,
      pltpu.SemaphoreType.DMA,
      pltpu.SemaphoreType.DMA,
      pltpu.SemaphoreType.DMA,
  ]
  if with_deg:
    scratch.append(pltpu.VMEM((N_PAD,), jnp.float32))
  if stage_table:
    scratch.append(pltpu.VMEM_SHARED((N_PAD, feat_half), jnp.float32))

  def body(xs, srcs, dsts, zrows, *rest):
    degp = tbl_sh = None
    if with_deg:
      zdeg, agg_out, deg_out = rest[0], rest[1], rest[2]
      scr = rest[3:]
    else:
      agg_out = rest[0]
      scr = rest[1:]
    (sa, da, sb, db, rows_a, rows_b, agg_sh,
     gsem_a, gsem_b, ssem_a, ssem_b, isem_a, isem_b) = scr[:13]
    scr = scr[13:]
    if with_deg:
      degp, scr = scr[0], scr[1:]
    if stage_table:
      tbl_sh = scr[0]
    c = lax.axis_index("c")
    s = lax.axis_index("s")
    row0 = s * ROWS_PER_TILE

    idx = ((sa, da, isem_a), (sb, db, isem_b))
    ring = ((rows_a, gsem_a, ssem_a), (rows_b, gsem_b, ssem_b))
    if mode == "feat":
      cbase = s * ngroups
    else:
      cbase = (c * N_TILES + s) * ngroups

    def start_idx(g, b):
      off = pl.multiple_of((cbase + g) * G, 8)
      sbuf, dbuf, isem = idx[b]
      pltpu.async_copy(srcs.at[pl.ds(off, G)], sbuf, isem)
      pltpu.async_copy(dsts.at[pl.ds(off, G)], dbuf, isem)

    def wait_idx(b):
      sbuf, dbuf, isem = idx[b]
      pltpu.make_async_copy(srcs.at[pl.ds(0, G)], sbuf, isem).wait()
      pltpu.make_async_copy(dsts.at[pl.ds(0, G)], dbuf, isem).wait()

    def table(idx_row):
      if stage_table:
        return tbl_sh.at[idx_row]
      if mode == "feat":
        return xs.at[c].at[idx_row]
      return xs.at[idx_row]

    def gather(idx_row, buf, sem):
      pltpu.async_copy(table(idx_row), buf, sem)

    def gwait(idx_row, buf, sem):
      pltpu.make_async_copy(table(idx_row), buf, sem).wait()

    def scat(idx_row, buf, ssem):
      pltpu.async_copy(buf, agg_sh.at[idx_row], ssem, add=True)

    def swait(idx_row, buf, ssem):
      pltpu.make_async_copy(buf, agg_sh.at[idx_row], ssem).wait()

    pltpu.sync_copy(zrows, agg_sh.at[pl.ds(row0, ROWS_PER_TILE)])
    if with_deg:
      pltpu.sync_copy(zdeg, degp)
    if stage_table:
      pltpu.sync_copy(xs.at[pl.ds(row0, ROWS_PER_TILE)],
                      tbl_sh.at[pl.ds(row0, ROWS_PER_TILE)])
    ones16 = jnp.ones((16,), jnp.float32)

    start_idx(0, 0)
    start_idx(1, 1)
    pltpu.sync_copy(zrows.at[pl.ds(0, CH)], rows_b)
    plsc.subcore_barrier()
    wait_idx(0)
    scat(da.at[0], rows_b, ssem_b)
    gather(sa.at[0], rows_a, gsem_a)

    def do_group(b):
      sbuf, dbuf, _ = idx[b]
      nsbuf, ndbuf, _ = idx[1 - b]
      for k in range(G):
        buf, gsem, ssem = ring[k % 2]
        nbuf, ngsem, nssem = ring[(k + 1) % 2]
        gwait(sbuf.at[k], buf, gsem)
        scat(dbuf.at[k], buf, ssem)
        swait(dbuf.at[k], nbuf, nssem)
        if k == G - 1:
          wait_idx(1 - b)
          gather(nsbuf.at[0], nbuf, ngsem)
        else:
          gather(sbuf.at[k + 1], nbuf, ngsem)
        if with_deg:
          @pl.when(c == b)
          def _():
            for j in range(CH // 16):
              dvec = dbuf[k, pl.ds(j * 16, 16)]
              plsc.addupdate_scatter(degp, [dvec], ones16)

    def pair(p, carry):
      do_group(0)
      start_idx(jnp.minimum(2 * p + 2, ngroups - 1), 0)
      do_group(1)
      start_idx(jnp.minimum(2 * p + 3, ngroups - 1), 1)
      return carry

    lax.fori_loop(0, ngroups // 2, pair, 0)
    gwait(sa.at[0], rows_a, gsem_a)
    swait(da.at[0], rows_b, ssem_b)
    wait_idx(1)
    plsc.subcore_barrier()

    pltpu.sync_copy(agg_sh.at[pl.ds(row0, ROWS_PER_TILE)],
                    agg_out.at[c].at[pl.ds(row0, ROWS_PER_TILE)])
    if with_deg:
      pltpu.sync_copy(degp, deg_out.at[c].at[s])

  params = pltpu.CompilerParams(use_tc_tiling_on_sc=tc_tiling,
                                needs_layout_passes=not with_deg)
  return pl.kernel(body, out_type=out_type, mesh=mesh, scratch_types=scratch,
                   compiler_params=params)


_R1 = 1000


def _dense1_body(agg_ref, x_ref, deg_ref, w_ref, wcls_ref, out_ref, y_ref):
  agg = agg_ref[...]
  h = jnp.concatenate([agg[0], agg[1]], axis=1) + x_ref[...]
  r = 1.0 / (deg_ref[...] + 1.0)
  h = h * r
  acc = lax.dot_general(h, w_ref[...], (((1,), (1,)), ((), ())),
                        preferred_element_type=jnp.float32)
  m = jnp.maximum(acc, 0.0)
  out_ref[...] = m
  y_ref[...] = lax.dot_general(m[:, :NHID], wcls_ref[...],
                               (((1,), (1,)), ((), ())),
                               preferred_element_type=jnp.float32)


def _dense1(agg, x, degc, wcat, w_cls):
  return pl.pallas_call(
      _dense1_body,
      grid=(N_NODES // _R1,),
      in_specs=[
          pl.BlockSpec((2, _R1, NHID), lambda i: (0, i, 0)),
          pl.BlockSpec((_R1, NFEAT), lambda i: (i, 0)),
          pl.BlockSpec((_R1, 1), lambda i: (i, 0)),
          pl.BlockSpec((2 * NHID, NFEAT), lambda i: (0, 0)),
          pl.BlockSpec((NCLASS, NHID), lambda i: (0, 0)),
      ],
      out_specs=[
          pl.BlockSpec((_R1, 2 * NHID), lambda i: (i, 0)),
          pl.BlockSpec((_R1, NCLASS), lambda i: (i, 0)),
      ],
      out_shape=[
          jax.ShapeDtypeStruct((N_NODES, 2 * NHID), jnp.float32),
          jax.ShapeDtypeStruct((N_NODES, NCLASS), jnp.float32),
      ],
  )(agg, x, degc, wcat, w_cls)


def _dense2_body(agg_ref, y_ref, deg_ref, out_ref):
  agg = agg_ref[...]
  h = agg[0] + agg[1] + y_ref[...]
  r = 1.0 / (deg_ref[...] + 1.0)
  out_ref[...] = jnp.maximum(h * r, 0.0)


def _dense2(agg2, y, degc):
  return pl.pallas_call(
      _dense2_body,
      grid=(N_NODES // _R1,),
      in_specs=[
          pl.BlockSpec((2, _R1, NCLASS), lambda i: (0, i, 0)),
          pl.BlockSpec((_R1, NCLASS), lambda i: (i, 0)),
          pl.BlockSpec((_R1, 1), lambda i: (i, 0)),
      ],
      out_specs=pl.BlockSpec((_R1, NCLASS), lambda i: (i, 0)),
      out_shape=jax.ShapeDtypeStruct((N_NODES, NCLASS), jnp.float32),
  )(agg2, y, degc)


_RB = 256


def _recons_body(a_ref, b_ref, out_ref):
  z = lax.dot_general(a_ref[...], b_ref[...], (((1,), (1,)), ((), ())),
                      preferred_element_type=jnp.float32)
  out_ref[...] = 1.0 / (1.0 + jnp.exp(-z))


def _recons(mu):
  return pl.pallas_call(
      _recons_body,
      grid=(pl.cdiv(N_NODES, _RB),),
      in_specs=[
          pl.BlockSpec((_RB, NHID), lambda i: (i, 0)),
          pl.BlockSpec((N_NODES, NHID), lambda i: (0, 0)),
      ],
      out_specs=pl.BlockSpec((_RB, N_NODES), lambda i: (i, 0)),
      out_shape=jax.ShapeDtypeStruct((N_NODES, N_NODES), jnp.float32),
  )(mu, mu)


def kernel(x, edge_index, W_mu, W_logvar, W_cls):
  src = edge_index[0].astype(jnp.int32)
  dst = edge_index[1].astype(jnp.int32)
  npad_e = E_PAD - N_EDGES
  pad_idx = jnp.full((npad_e,), N_PAD - 1, jnp.int32)
  src_p = jnp.concatenate([src, pad_idx]).reshape(E_PAD // CH, CH)
  dst_p = jnp.concatenate([dst, pad_idx]).reshape(E_PAD // CH, CH)

  x_pad = jnp.zeros((N_PAD, NFEAT), jnp.float32).at[:N_NODES].set(x)
  xs = jnp.stack([x_pad[:, :NHID], x_pad[:, NHID:]])
  zrows1 = jnp.zeros((ROWS_PER_TILE, NHID), jnp.float32)
  zdeg = jnp.zeros((N_PAD,), jnp.float32)

  sc1 = _make_sc_agg(NHID, with_deg=True, mode="feat", group=8,
                     tc_tiling=False)
  agg, deg = sc1(xs, src_p, dst_p, zrows1, zdeg)
  degq = jnp.sum(deg, axis=(0, 1))[:N_NODES, None]

  wcat = jnp.concatenate([W_mu, W_logvar], axis=0)
  ml, y = _dense1(agg, x, degq, wcat, W_cls)
  mu = ml[:, :NHID]
  logvar = ml[:, NHID:]

  y_pad = jnp.zeros((N_PAD, NCLASS), jnp.float32).at[:N_NODES].set(y)
  zrows2 = jnp.zeros((ROWS_PER_TILE, NCLASS), jnp.float32)

  sc2 = _make_sc_agg(NCLASS, with_deg=False, mode="edge", group=8,
                     tc_tiling=False, stage_table=True)
  agg2 = sc2(y_pad, src_p, dst_p, zrows2)
  if isinstance(agg2, (tuple, list)):
    agg2 = agg2[0]

  rst = _dense2(agg2, y, degq)
  recons = _recons(mu)
  return (rst, recons, mu, logvar)


# speedup vs baseline: 2.2685x; 1.8451x over previous
import functools

import jax
import jax.numpy as jnp
from jax import lax
from jax.experimental import pallas as pl
from jax.experimental.pallas import tpu as pltpu
from jax.experimental.pallas import tpu_sc as plsc

N_NODES = 10000
N_EDGES = 320000
NFEAT = 256
NHID = 128
NCLASS = 64

N_TILES = 16
CH = 128
N_PAD = 10240
ROWS_PER_TILE = N_PAD // N_TILES
E_PAD = 327680
CHUNKS = E_PAD // N_TILES // CH


def _make_sc_agg(feat_half, with_deg, mode, group, tc_tiling=True,
                 stage_table=False):
  mesh = plsc.VectorSubcoreMesh(core_axis_name="c", subcore_axis_name="s")
  G = group
  chunks = CHUNKS if mode == "feat" else CHUNKS // 2
  ngroups = chunks // G
  assert ngroups * G == chunks and ngroups % 2 == 0
  out_type = [jax.ShapeDtypeStruct((2, N_PAD, feat_half), jnp.float32)]
  if with_deg:
    out_type.append(jax.ShapeDtypeStruct((2, N_TILES, N_PAD), jnp.float32))
  scratch = [
      pltpu.VMEM((G, CH), jnp.int32),
      pltpu.VMEM((G, CH), jnp.int32),
      pltpu.VMEM((G, CH), jnp.int32),
      pltpu.VMEM((G, CH), jnp.int32),
      pltpu.VMEM((CH, feat_half), jnp.float32),
      pltpu.VMEM((CH, feat_half), jnp.float32),
      pltpu.VMEM_SHARED((N_PAD, feat_half), jnp.float32),
      pltpu.SemaphoreType.DMA,
      pltpu.SemaphoreType.DMA,
      pltpu.SemaphoreType.DMA,
      pltpu.SemaphoreType.DMA,
      pltpu.SemaphoreType.DMA,
      pltpu.SemaphoreType.DMA,
  ]
  if with_deg:
    scratch.append(pltpu.VMEM((N_PAD,), jnp.float32))
  if stage_table:
    scratch.append(pltpu.VMEM_SHARED((N_PAD, feat_half), jnp.float32))

  def body(xs, srcs, dsts, zrows, *rest):
    degp = tbl_sh = None
    if with_deg:
      zdeg, agg_out, deg_out = rest[0], rest[1], rest[2]
      scr = rest[3:]
    else:
      agg_out = rest[0]
      scr = rest[1:]
    (sa, da, sb, db, rows_a, rows_b, agg_sh,
     gsem_a, gsem_b, ssem_a, ssem_b, isem_a, isem_b) = scr[:13]
    scr = scr[13:]
    if with_deg:
      degp, scr = scr[0], scr[1:]
    if stage_table:
      tbl_sh = scr[0]
    c = lax.axis_index("c")
    s = lax.axis_index("s")
    row0 = s * ROWS_PER_TILE

    idx = ((sa, da, isem_a), (sb, db, isem_b))
    ring = ((rows_a, gsem_a, ssem_a), (rows_b, gsem_b, ssem_b))
    if mode == "feat":
      cbase = s * ngroups
    else:
      cbase = (c * N_TILES + s) * ngroups

    def start_idx(g, b):
      off = pl.multiple_of((cbase + g) * G, 8)
      sbuf, dbuf, isem = idx[b]
      pltpu.async_copy(srcs.at[pl.ds(off, G)], sbuf, isem)
      pltpu.async_copy(dsts.at[pl.ds(off, G)], dbuf, isem)

    def wait_idx(b):
      sbuf, dbuf, isem = idx[b]
      pltpu.make_async_copy(srcs.at[pl.ds(0, G)], sbuf, isem).wait()
      pltpu.make_async_copy(dsts.at[pl.ds(0, G)], dbuf, isem).wait()

    def table(idx_row):
      if stage_table:
        return tbl_sh.at[idx_row]
      if mode == "feat":
        return xs.at[c].at[idx_row]
      return xs.at[idx_row]

    def gather(idx_row, buf, sem):
      pltpu.async_copy(table(idx_row), buf, sem)

    def gwait(idx_row, buf, sem):
      pltpu.make_async_copy(table(idx_row), buf, sem).wait()

    def scat(idx_row, buf, ssem):
      pltpu.async_copy(buf, agg_sh.at[idx_row], ssem, add=True)

    def swait(idx_row, buf, ssem):
      pltpu.make_async_copy(buf, agg_sh.at[idx_row], ssem).wait()

    pltpu.sync_copy(zrows, agg_sh.at[pl.ds(row0, ROWS_PER_TILE)])
    if with_deg:
      pltpu.sync_copy(zdeg, degp)
    if stage_table:
      pltpu.sync_copy(xs.at[pl.ds(row0, ROWS_PER_TILE)],
                      tbl_sh.at[pl.ds(row0, ROWS_PER_TILE)])
    ones16 = jnp.ones((16,), jnp.float32)

    start_idx(0, 0)
    start_idx(1, 1)
    pltpu.sync_copy(zrows.at[pl.ds(0, CH)], rows_b)
    plsc.subcore_barrier()
    wait_idx(0)
    scat(da.at[0], rows_b, ssem_b)
    gather(sa.at[0], rows_a, gsem_a)

    def do_group(b):
      sbuf, dbuf, _ = idx[b]
      nsbuf, ndbuf, _ = idx[1 - b]
      for k in range(G):
        buf, gsem, ssem = ring[k % 2]
        nbuf, ngsem, nssem = ring[(k + 1) % 2]
        gwait(sbuf.at[k], buf, gsem)
        scat(dbuf.at[k], buf, ssem)
        swait(dbuf.at[k], nbuf, nssem)
        if k == G - 1:
          wait_idx(1 - b)
          gather(nsbuf.at[0], nbuf, ngsem)
        else:
          gather(sbuf.at[k + 1], nbuf, ngsem)
        if with_deg:
          @pl.when(c == b)
          def _():
            for j in range(CH // 16):
              dvec = dbuf[k, pl.ds(j * 16, 16)]
              plsc.addupdate_scatter(degp, [dvec], ones16)

    def pair(p, carry):
      do_group(0)
      start_idx(jnp.minimum(2 * p + 2, ngroups - 1), 0)
      do_group(1)
      start_idx(jnp.minimum(2 * p + 3, ngroups - 1), 1)
      return carry

    lax.fori_loop(0, ngroups // 2, pair, 0)
    gwait(sa.at[0], rows_a, gsem_a)
    swait(da.at[0], rows_b, ssem_b)
    wait_idx(1)
    plsc.subcore_barrier()

    pltpu.sync_copy(agg_sh.at[pl.ds(row0, ROWS_PER_TILE)],
                    agg_out.at[c].at[pl.ds(row0, ROWS_PER_TILE)])
    if with_deg:
      pltpu.sync_copy(degp, deg_out.at[c].at[s])

  params = pltpu.CompilerParams(use_tc_tiling_on_sc=tc_tiling,
                                needs_layout_passes=not with_deg)
  return pl.kernel(body, out_type=out_type, mesh=mesh, scratch_types=scratch,
                   compiler_params=params)


_R1 = 1000


def _dense1_body(agg_ref, x_ref, deg_ref, w_ref, wcls_ref, out_ref, y_ref):
  agg = agg_ref[...]
  h = jnp.concatenate([agg[0], agg[1]], axis=1) + x_ref[...]
  r = 1.0 / (deg_ref[...] + 1.0)
  h = h * r
  acc = lax.dot_general(h, w_ref[...], (((1,), (1,)), ((), ())),
                        preferred_element_type=jnp.float32)
  m = jnp.maximum(acc, 0.0)
  out_ref[...] = m
  y_ref[...] = lax.dot_general(m[:, :NHID], wcls_ref[...],
                               (((1,), (1,)), ((), ())),
                               preferred_element_type=jnp.float32)


def _dense1(agg, x, degc, wcat, w_cls):
  return pl.pallas_call(
      _dense1_body,
      grid=(N_NODES // _R1,),
      in_specs=[
          pl.BlockSpec((2, _R1, NHID), lambda i: (0, i, 0)),
          pl.BlockSpec((_R1, NFEAT), lambda i: (i, 0)),
          pl.BlockSpec((_R1, 1), lambda i: (i, 0)),
          pl.BlockSpec((2 * NHID, NFEAT), lambda i: (0, 0)),
          pl.BlockSpec((NCLASS, NHID), lambda i: (0, 0)),
      ],
      out_specs=[
          pl.BlockSpec((_R1, 2 * NHID), lambda i: (i, 0)),
          pl.BlockSpec((_R1, NCLASS), lambda i: (i, 0)),
      ],
      out_shape=[
          jax.ShapeDtypeStruct((N_NODES, 2 * NHID), jnp.float32),
          jax.ShapeDtypeStruct((N_NODES, NCLASS), jnp.float32),
      ],
  )(agg, x, degc, wcat, w_cls)


def _dense2_body(agg_ref, y_ref, deg_ref, out_ref):
  agg = agg_ref[...]
  h = agg[0] + agg[1] + y_ref[...]
  r = 1.0 / (deg_ref[...] + 1.0)
  out_ref[...] = jnp.maximum(h * r, 0.0)


def _dense2(agg2, y, degc):
  return pl.pallas_call(
      _dense2_body,
      grid=(N_NODES // _R1,),
      in_specs=[
          pl.BlockSpec((2, _R1, NCLASS), lambda i: (0, i, 0)),
          pl.BlockSpec((_R1, NCLASS), lambda i: (i, 0)),
          pl.BlockSpec((_R1, 1), lambda i: (i, 0)),
      ],
      out_specs=pl.BlockSpec((_R1, NCLASS), lambda i: (i, 0)),
      out_shape=jax.ShapeDtypeStruct((N_NODES, NCLASS), jnp.float32),
  )(agg2, y, degc)


_RB = 256


def _recons_body(a_ref, b_ref, out_ref):
  z = lax.dot_general(a_ref[...], b_ref[...], (((1,), (1,)), ((), ())),
                      preferred_element_type=jnp.float32)
  out_ref[...] = 1.0 / (1.0 + jnp.exp(-z))


def _recons(mu):
  return pl.pallas_call(
      _recons_body,
      grid=(pl.cdiv(N_NODES, _RB),),
      in_specs=[
          pl.BlockSpec((_RB, NHID), lambda i: (i, 0)),
          pl.BlockSpec((N_NODES, NHID), lambda i: (0, 0)),
      ],
      out_specs=pl.BlockSpec((_RB, N_NODES), lambda i: (i, 0)),
      out_shape=jax.ShapeDtypeStruct((N_NODES, N_NODES), jnp.float32),
  )(mu, mu)


def kernel(x, edge_index, W_mu, W_logvar, W_cls):
  src = edge_index[0].astype(jnp.int32)
  dst = edge_index[1].astype(jnp.int32)
  npad_e = E_PAD - N_EDGES
  pad_idx = N_NODES + (jnp.arange(npad_e, dtype=jnp.int32)
                       % (N_PAD - N_NODES))
  src_p = jnp.concatenate([src, pad_idx]).reshape(E_PAD // CH, CH)
  dst_p = jnp.concatenate([dst, pad_idx]).reshape(E_PAD // CH, CH)

  x_pad = jnp.zeros((N_PAD, NFEAT), jnp.float32).at[:N_NODES].set(x)
  xs = jnp.stack([x_pad[:, :NHID], x_pad[:, NHID:]])
  zrows1 = jnp.zeros((ROWS_PER_TILE, NHID), jnp.float32)
  zdeg = jnp.zeros((N_PAD,), jnp.float32)

  sc1 = _make_sc_agg(NHID, with_deg=True, mode="feat", group=8,
                     tc_tiling=False)
  agg, deg = sc1(xs, src_p, dst_p, zrows1, zdeg)
  degq = jnp.sum(deg, axis=(0, 1))[:N_NODES, None]

  wcat = jnp.concatenate([W_mu, W_logvar], axis=0)
  ml, y = _dense1(agg, x, degq, wcat, W_cls)
  mu = ml[:, :NHID]
  logvar = ml[:, NHID:]

  y_pad = jnp.zeros((N_PAD, NCLASS), jnp.float32).at[:N_NODES].set(y)
  zrows2 = jnp.zeros((ROWS_PER_TILE, NCLASS), jnp.float32)

  sc2 = _make_sc_agg(NCLASS, with_deg=False, mode="edge", group=8,
                     tc_tiling=False, stage_table=True)
  agg2 = sc2(y_pad, src_p, dst_p, zrows2)
  if isinstance(agg2, (tuple, list)):
    agg2 = agg2[0]

  rst = _dense2(agg2, y, degq)
  recons = _recons(mu)
  return (rst, recons, mu, logvar)


# speedup vs baseline: 2.3393x; 1.0312x over previous
import functools

import jax
import jax.numpy as jnp
from jax import lax
from jax.experimental import pallas as pl
from jax.experimental.pallas import tpu as pltpu
from jax.experimental.pallas import tpu_sc as plsc

N_NODES = 10000
N_EDGES = 320000
NFEAT = 256
NHID = 128
NCLASS = 64

N_TILES = 16
CH = 128
N_PAD = 10240
ROWS_PER_TILE = N_PAD // N_TILES
E_PAD = 327680
CHUNKS = E_PAD // N_TILES // CH


def _make_sc_agg(feat_half, with_deg, mode, group, tc_tiling=True,
                 stage_table=False):
  mesh = plsc.VectorSubcoreMesh(core_axis_name="c", subcore_axis_name="s")
  G = group
  chunks = CHUNKS if mode == "feat" else CHUNKS // 2
  ngroups = chunks // G
  assert ngroups * G == chunks and ngroups % 2 == 0
  out_type = [jax.ShapeDtypeStruct((2, N_PAD, feat_half), jnp.float32)]
  if with_deg:
    out_type.append(jax.ShapeDtypeStruct((2, N_TILES, N_PAD), jnp.float32))
  scratch = [
      pltpu.VMEM((G, CH), jnp.int32),
      pltpu.VMEM((G, CH), jnp.int32),
      pltpu.VMEM((G, CH), jnp.int32),
      pltpu.VMEM((G, CH), jnp.int32),
      pltpu.VMEM((CH, feat_half), jnp.float32),
      pltpu.VMEM((CH, feat_half), jnp.float32),
      pltpu.VMEM_SHARED((N_PAD, feat_half), jnp.float32),
      pltpu.SemaphoreType.DMA,
      pltpu.SemaphoreType.DMA,
      pltpu.SemaphoreType.DMA,
      pltpu.SemaphoreType.DMA,
      pltpu.SemaphoreType.DMA,
      pltpu.SemaphoreType.DMA,
  ]
  if with_deg:
    scratch.append(pltpu.VMEM((N_PAD,), jnp.float32))
  if stage_table:
    scratch.append(pltpu.VMEM_SHARED((N_PAD, feat_half), jnp.float32))

  def body(xs, srcs, dsts, zrows, *rest):
    degp = tbl_sh = None
    if with_deg:
      zdeg, agg_out, deg_out = rest[0], rest[1], rest[2]
      scr = rest[3:]
    else:
      agg_out = rest[0]
      scr = rest[1:]
    (sa, da, sb, db, rows_a, rows_b, agg_sh,
     gsem_a, gsem_b, ssem_a, ssem_b, isem_a, isem_b) = scr[:13]
    scr = scr[13:]
    if with_deg:
      degp, scr = scr[0], scr[1:]
    if stage_table:
      tbl_sh = scr[0]
    c = lax.axis_index("c")
    s = lax.axis_index("s")
    row0 = s * ROWS_PER_TILE

    idx = ((sa, da, isem_a), (sb, db, isem_b))
    ring = ((rows_a, gsem_a, ssem_a), (rows_b, gsem_b, ssem_b))
    if mode == "feat":
      cbase = s * ngroups
    else:
      cbase = (c * N_TILES + s) * ngroups

    def start_idx(g, b):
      off = pl.multiple_of((cbase + g) * G, 8)
      sbuf, dbuf, isem = idx[b]
      pltpu.async_copy(srcs.at[pl.ds(off, G)], sbuf, isem)
      pltpu.async_copy(dsts.at[pl.ds(off, G)], dbuf, isem)

    def wait_idx(b):
      sbuf, dbuf, isem = idx[b]
      pltpu.make_async_copy(srcs.at[pl.ds(0, G)], sbuf, isem).wait()
      pltpu.make_async_copy(dsts.at[pl.ds(0, G)], dbuf, isem).wait()

    def table(idx_row):
      if stage_table:
        return tbl_sh.at[idx_row]
      if mode == "feat":
        return xs.at[c].at[idx_row]
      return xs.at[idx_row]

    def gather(idx_row, buf, sem):
      pltpu.async_copy(table(idx_row), buf, sem)

    def gwait(idx_row, buf, sem):
      pltpu.make_async_copy(table(idx_row), buf, sem).wait()

    def scat(idx_row, buf, ssem):
      pltpu.async_copy(buf, agg_sh.at[idx_row], ssem, add=True)

    def swait(idx_row, buf, ssem):
      pltpu.make_async_copy(buf, agg_sh.at[idx_row], ssem).wait()

    pltpu.sync_copy(zrows, agg_sh.at[pl.ds(row0, ROWS_PER_TILE)])
    if with_deg:
      pltpu.sync_copy(zdeg, degp)
    if stage_table:
      pltpu.sync_copy(xs.at[pl.ds(row0, ROWS_PER_TILE)],
                      tbl_sh.at[pl.ds(row0, ROWS_PER_TILE)])
    ones16 = jnp.ones((16,), jnp.float32)

    start_idx(0, 0)
    start_idx(1, 1)
    pltpu.sync_copy(zrows.at[pl.ds(0, CH)], rows_b)
    plsc.subcore_barrier()
    wait_idx(0)
    scat(da.at[0], rows_b, ssem_b)
    gather(sa.at[0], rows_a, gsem_a)

    def do_group(b):
      sbuf, dbuf, _ = idx[b]
      nsbuf, ndbuf, _ = idx[1 - b]
      for k in range(G):
        buf, gsem, ssem = ring[k % 2]
        nbuf, ngsem, nssem = ring[(k + 1) % 2]
        gwait(sbuf.at[k], buf, gsem)
        scat(dbuf.at[k], buf, ssem)
        swait(dbuf.at[k], nbuf, nssem)
        if k == G - 1:
          wait_idx(1 - b)
          gather(nsbuf.at[0], nbuf, ngsem)
        else:
          gather(sbuf.at[k + 1], nbuf, ngsem)
        if with_deg:
          @pl.when(c == b)
          def _():
            for j in range(CH // 16):
              dvec = dbuf[k, pl.ds(j * 16, 16)]
              plsc.addupdate_scatter(degp, [dvec], ones16)

    def pair(p, carry):
      do_group(0)
      start_idx(jnp.minimum(2 * p + 2, ngroups - 1), 0)
      do_group(1)
      start_idx(jnp.minimum(2 * p + 3, ngroups - 1), 1)
      return carry

    lax.fori_loop(0, ngroups // 2, pair, 0)
    gwait(sa.at[0], rows_a, gsem_a)
    swait(da.at[0], rows_b, ssem_b)
    wait_idx(1)
    plsc.subcore_barrier()

    pltpu.sync_copy(agg_sh.at[pl.ds(row0, ROWS_PER_TILE)],
                    agg_out.at[c].at[pl.ds(row0, ROWS_PER_TILE)])
    if with_deg:
      pltpu.sync_copy(degp, deg_out.at[c].at[s])

  params = pltpu.CompilerParams(use_tc_tiling_on_sc=tc_tiling,
                                needs_layout_passes=not with_deg)
  return pl.kernel(body, out_type=out_type, mesh=mesh, scratch_types=scratch,
                   compiler_params=params)


_R1 = 1000


def _dense1_body(agg_ref, x_ref, deg_ref, w_ref, wcls_ref, out_ref, y_ref):
  agg = agg_ref[...]
  h = jnp.concatenate([agg[0], agg[1]], axis=1) + x_ref[...]
  r = 1.0 / (deg_ref[...] + 1.0)
  h = h * r
  acc = lax.dot_general(h, w_ref[...], (((1,), (1,)), ((), ())),
                        preferred_element_type=jnp.float32)
  m = jnp.maximum(acc, 0.0)
  out_ref[...] = m
  y_ref[...] = lax.dot_general(m[:, :NHID], wcls_ref[...],
                               (((1,), (1,)), ((), ())),
                               preferred_element_type=jnp.float32)


def _dense1(agg, x, degc, wcat, w_cls):
  return pl.pallas_call(
      _dense1_body,
      grid=(N_NODES // _R1,),
      in_specs=[
          pl.BlockSpec((2, _R1, NHID), lambda i: (0, i, 0)),
          pl.BlockSpec((_R1, NFEAT), lambda i: (i, 0)),
          pl.BlockSpec((_R1, 1), lambda i: (i, 0)),
          pl.BlockSpec((2 * NHID, NFEAT), lambda i: (0, 0)),
          pl.BlockSpec((NCLASS, NHID), lambda i: (0, 0)),
      ],
      out_specs=[
          pl.BlockSpec((_R1, 2 * NHID), lambda i: (i, 0)),
          pl.BlockSpec((_R1, NCLASS), lambda i: (i, 0)),
      ],
      out_shape=[
          jax.ShapeDtypeStruct((N_NODES, 2 * NHID), jnp.float32),
          jax.ShapeDtypeStruct((N_NODES, NCLASS), jnp.float32),
      ],
  )(agg, x, degc, wcat, w_cls)


def _dense2_body(agg_ref, y_ref, deg_ref, out_ref):
  agg = agg_ref[...]
  h = agg[0] + agg[1] + y_ref[...]
  r = 1.0 / (deg_ref[...] + 1.0)
  out_ref[...] = jnp.maximum(h * r, 0.0)


def _dense2(agg2, y, degc):
  return pl.pallas_call(
      _dense2_body,
      grid=(N_NODES // _R1,),
      in_specs=[
          pl.BlockSpec((2, _R1, NCLASS), lambda i: (0, i, 0)),
          pl.BlockSpec((_R1, NCLASS), lambda i: (i, 0)),
          pl.BlockSpec((_R1, 1), lambda i: (i, 0)),
      ],
      out_specs=pl.BlockSpec((_R1, NCLASS), lambda i: (i, 0)),
      out_shape=jax.ShapeDtypeStruct((N_NODES, NCLASS), jnp.float32),
  )(agg2, y, degc)


_RB = 512


def _recons_body(a_ref, b_ref, out_ref):
  z = lax.dot_general(a_ref[...], b_ref[...], (((1,), (1,)), ((), ())),
                      preferred_element_type=jnp.float32)
  out_ref[...] = 1.0 / (1.0 + jnp.exp(-z))


def _recons(mu):
  return pl.pallas_call(
      _recons_body,
      grid=(pl.cdiv(N_NODES, _RB),),
      in_specs=[
          pl.BlockSpec((_RB, NHID), lambda i: (i, 0)),
          pl.BlockSpec((N_NODES, NHID), lambda i: (0, 0)),
      ],
      out_specs=pl.BlockSpec((_RB, N_NODES), lambda i: (i, 0)),
      out_shape=jax.ShapeDtypeStruct((N_NODES, N_NODES), jnp.float32),
  )(mu, mu)


def kernel(x, edge_index, W_mu, W_logvar, W_cls):
  src = edge_index[0].astype(jnp.int32)
  dst = edge_index[1].astype(jnp.int32)
  npad_e = E_PAD - N_EDGES
  pad_idx = N_NODES + (jnp.arange(npad_e, dtype=jnp.int32)
                       % (N_PAD - N_NODES))
  src_p = jnp.concatenate([src, pad_idx]).reshape(E_PAD // CH, CH)
  dst_p = jnp.concatenate([dst, pad_idx]).reshape(E_PAD // CH, CH)

  x_pad = jnp.zeros((N_PAD, NFEAT), jnp.float32).at[:N_NODES].set(x)
  xs = jnp.stack([x_pad[:, :NHID], x_pad[:, NHID:]])
  zrows1 = jnp.zeros((ROWS_PER_TILE, NHID), jnp.float32)
  zdeg = jnp.zeros((N_PAD,), jnp.float32)

  sc1 = _make_sc_agg(NHID, with_deg=True, mode="feat", group=8,
                     tc_tiling=False)
  agg, deg = sc1(xs, src_p, dst_p, zrows1, zdeg)
  degq = jnp.sum(deg, axis=(0, 1))[:N_NODES, None]

  wcat = jnp.concatenate([W_mu, W_logvar], axis=0)
  ml, y = _dense1(agg, x, degq, wcat, W_cls)
  mu = ml[:, :NHID]
  logvar = ml[:, NHID:]

  y_pad = jnp.zeros((N_PAD, NCLASS), jnp.float32).at[:N_NODES].set(y)
  zrows2 = jnp.zeros((ROWS_PER_TILE, NCLASS), jnp.float32)

  sc2 = _make_sc_agg(NCLASS, with_deg=False, mode="edge", group=8,
                     tc_tiling=False, stage_table=True)
  agg2 = sc2(y_pad, src_p, dst_p, zrows2)
  if isinstance(agg2, (tuple, list)):
    agg2 = agg2[0]

  rst = _dense2(agg2, y, degq)
  recons = _recons(mu)
  return (rst, recons, mu, logvar)
